# baseline (speedup 1.0000x reference)
import functools
import jax
import jax.numpy as jnp
from jax import lax
from jax.experimental import pallas as pl
from jax.experimental.pallas import tpu as pltpu
from jax.experimental.pallas import tpu_sc as plsc

_K = 16
_L = 16


def _sc_spline(x_hbm, table_hbm, out_hbm, x_v, idx_v, w_v, rows_v, ostage_v, sem):
    NC = 2
    wid = lax.axis_index("s") * NC + lax.axis_index("c")
    D = x_hbm.shape[1]
    rows_per_w = x_hbm.shape[0] // 32
    nchunk = D // _L
    base = wid * rows_per_w
    pltpu.sync_copy(x_hbm.at[pl.ds(base, rows_per_w)], x_v)

    def row_body(r, carry):
        for c in range(nchunk):
            xv = x_v[r, pl.ds(c * _L, _L)]
            xc = jnp.minimum(jnp.maximum(xv, -1.0), 1.0)
            t = (xc + 1.0) * 7.5
            li = t.astype(jnp.int32)
            li = jnp.minimum(jnp.maximum(li, 0), _K - 2)
            w = t - li.astype(jnp.float32)
            dvec = jax.lax.iota(jnp.int32, _L) + c * _L
            idx_v[pl.ds(c * _L, _L)] = dvec * _K + li
            idx_v[pl.ds(D + c * _L, _L)] = dvec * _K + li + 1
            w_v[pl.ds(c * _L, _L)] = w
        copies = []
        for j in range((2 * D) // 128):
            copies.append(pltpu.async_copy(
                table_hbm.at[idx_v.at[pl.ds(j * 128, 128)]],
                rows_v.at[pl.ds(j * 128, 128)], sem))
        for cp in copies:
            cp.wait()
        def c_body(c, acc):
            wchunk = w_v[pl.ds(c * _L, _L)]
            out = list(acc)
            for dd in range(_L):
                d = c * _L + dd
                wd = wchunk.at[jnp.full((_L,), dd, jnp.int32)].get(
                    mode="promise_in_bounds")
                for j in range(8):
                    lv = rows_v[d, pl.ds(j * _L, _L)]
                    rv = rows_v[D + d, pl.ds(j * _L, _L)]
                    out[j] = out[j] + (lv + wd * (rv - lv))
            return tuple(out)
        acc0 = tuple(jnp.zeros((_L,), jnp.float32) for _ in range(8))
        acc = lax.fori_loop(0, nchunk, c_body, acc0)
        for j in range(8):
            ostage_v[r, pl.ds(j * _L, _L)] = acc[j]
        return carry

    lax.fori_loop(0, rows_per_w, row_body, 0)
    pltpu.sync_copy(ostage_v, out_hbm.at[pl.ds(base, rows_per_w)])


def _skip_body(x_ref, sw_ref, sb_ref, sp_ref, o_ref):
    xc = jnp.clip(x_ref[...], -1.0, 1.0)
    o_ref[...] = (jax.lax.dot(xc, sw_ref[...], preferred_element_type=jnp.float32)
                  + sb_ref[...] + sp_ref[...])


def kernel(x, values, skip_w, skip_b, grid):
    B, D = x.shape
    O = values.shape[0]
    tableT = jnp.transpose(values, (1, 2, 0)).reshape(D * _K, O)
    rows_per_w = B // 32

    mesh = plsc.VectorSubcoreMesh(core_axis_name="c", subcore_axis_name="s")
    spline = pl.kernel(
        _sc_spline,
        mesh=mesh,
        out_type=jax.ShapeDtypeStruct((B, O), jnp.float32),
        scratch_types=[
            pltpu.VMEM((rows_per_w, D), jnp.float32),
            pltpu.VMEM((2 * D,), jnp.int32),
            pltpu.VMEM((D,), jnp.float32),
            pltpu.VMEM((2 * D, O), jnp.float32),
            pltpu.VMEM((rows_per_w, O), jnp.float32),
            pltpu.SemaphoreType.DMA,
        ],
    )(x, tableT)

    sw = skip_w.T
    sb = skip_b.reshape(1, O)
    return pl.pallas_call(
        _skip_body,
        out_shape=jax.ShapeDtypeStruct((B, O), jnp.float32),
        in_specs=[pl.BlockSpec(memory_space=pltpu.VMEM)] * 4,
        out_specs=pl.BlockSpec(memory_space=pltpu.VMEM),
    )(x, sw, sb, spline)


# speedup vs baseline: 1.0011x; 1.0011x over previous
import functools
import jax
import jax.numpy as jnp
from jax import lax
from jax.experimental import pallas as pl
from jax.experimental.pallas import tpu as pltpu
from jax.experimental.pallas import tpu_sc as plsc

_K = 16
_L = 16


def _sc_spline(x_hbm, table_hbm, out_hbm, x_v, idx_v, w_v, rows_v, ostage_v, sem):
    NC = 2
    wid = lax.axis_index("s") * NC + lax.axis_index("c")
    D = x_hbm.shape[1]
    rows_per_w = x_hbm.shape[0] // 32
    nchunk = D // _L
    base = wid * rows_per_w
    pltpu.sync_copy(x_hbm.at[pl.ds(base, rows_per_w)], x_v)

    def row_body(r, carry):
        for c in range(nchunk):
            xv = x_v[r, pl.ds(c * _L, _L)]
            xc = jnp.minimum(jnp.maximum(xv, -1.0), 1.0)
            t = (xc + 1.0) * 7.5
            li = t.astype(jnp.int32)
            li = jnp.minimum(jnp.maximum(li, 0), _K - 2)
            w = t - li.astype(jnp.float32)
            dvec = jax.lax.iota(jnp.int32, _L) + c * _L
            idx_v[pl.ds(c * _L, _L)] = dvec * _K + li
            idx_v[pl.ds(D + c * _L, _L)] = dvec * _K + li + 1
            w_v[pl.ds(c * _L, _L)] = w
        copies = []
        for j in range((2 * D) // 128):
            copies.append(pltpu.async_copy(
                table_hbm.at[idx_v.at[pl.ds(j * 128, 128)]],
                rows_v.at[pl.ds(j * 128, 128)], sem))
        for cp in copies:
            cp.wait()
        def c_body(c, acc):
            wchunk = w_v[pl.ds(c * _L, _L)]
            out = list(acc)
            for dd in range(_L):
                d = c * _L + dd
                wd = wchunk.at[jnp.full((_L,), dd, jnp.int32)].get(
                    mode="promise_in_bounds")
                for j in range(8):
                    lv = rows_v[d, pl.ds(j * _L, _L)]
                    rv = rows_v[D + d, pl.ds(j * _L, _L)]
                    out[j] = out[j] + (lv + wd * (rv - lv))
            return tuple(out)
        acc0 = tuple(jnp.zeros((_L,), jnp.float32) for _ in range(8))
        acc = plsc.parallel_loop(0, nchunk, 1, unroll=2, carry=acc0)(c_body)
        for j in range(8):
            ostage_v[r, pl.ds(j * _L, _L)] = acc[j]
        return carry

    lax.fori_loop(0, rows_per_w, row_body, 0)
    pltpu.sync_copy(ostage_v, out_hbm.at[pl.ds(base, rows_per_w)])


def _skip_body(x_ref, sw_ref, sb_ref, sp_ref, o_ref):
    xc = jnp.clip(x_ref[...], -1.0, 1.0)
    o_ref[...] = (jax.lax.dot(xc, sw_ref[...], preferred_element_type=jnp.float32)
                  + sb_ref[...] + sp_ref[...])


def kernel(x, values, skip_w, skip_b, grid):
    B, D = x.shape
    O = values.shape[0]
    tableT = jnp.transpose(values, (1, 2, 0)).reshape(D * _K, O)
    rows_per_w = B // 32

    mesh = plsc.VectorSubcoreMesh(core_axis_name="c", subcore_axis_name="s")
    spline = pl.kernel(
        _sc_spline,
        mesh=mesh,
        out_type=jax.ShapeDtypeStruct((B, O), jnp.float32),
        scratch_types=[
            pltpu.VMEM((rows_per_w, D), jnp.float32),
            pltpu.VMEM((2 * D,), jnp.int32),
            pltpu.VMEM((D,), jnp.float32),
            pltpu.VMEM((2 * D, O), jnp.float32),
            pltpu.VMEM((rows_per_w, O), jnp.float32),
            pltpu.SemaphoreType.DMA,
        ],
    )(x, tableT)

    sw = skip_w.T
    sb = skip_b.reshape(1, O)
    return pl.pallas_call(
        _skip_body,
        out_shape=jax.ShapeDtypeStruct((B, O), jnp.float32),
        in_specs=[pl.BlockSpec(memory_space=pltpu.VMEM)] * 4,
        out_specs=pl.BlockSpec(memory_space=pltpu.VMEM),
    )(x, sw, sb, spline)


# speedup vs baseline: 7.0914x; 7.0838x over previous
import jax
import jax.numpy as jnp
from jax import lax
from jax.experimental import pallas as pl
from jax.experimental.pallas import tpu as pltpu
from jax.experimental.pallas import tpu_sc as plsc

_K = 16
_L = 16
_NSC = 32


def _sc_spline(x_hbm, table_hbm, out_hbm, x_v, idx_v, w_v, rows_v, ostage_v, sem):
    wid = lax.axis_index("s") * 2 + lax.axis_index("c")
    D = x_hbm.shape[1]
    nchunk = D // _L
    pltpu.sync_copy(x_hbm.at[pl.ds(wid, 1)], x_v)
    for c in range(nchunk):
        xv = x_v[0, pl.ds(c * _L, _L)]
        xc = jnp.minimum(jnp.maximum(xv, -1.0), 1.0)
        t = (xc + 1.0) * 7.5
        li = t.astype(jnp.int32)
        li = jnp.minimum(jnp.maximum(li, 0), _K - 2)
        w = t - li.astype(jnp.float32)
        dvec = jax.lax.iota(jnp.int32, _L) + c * _L
        idx_v[pl.ds(c * _L, _L)] = dvec * _K + li
        idx_v[pl.ds(D + c * _L, _L)] = dvec * _K + li + 1
        w_v[pl.ds(c * _L, _L)] = w
    copies = []
    for j in range((2 * D) // 128):
        copies.append(pltpu.async_copy(
            table_hbm.at[idx_v.at[pl.ds(j * 128, 128)]],
            rows_v.at[pl.ds(j * 128, 128)], sem))
    for cp in copies:
        cp.wait()

    def c_body(c, acc):
        wchunk = w_v[pl.ds(c * _L, _L)]
        out = list(acc)
        for dd in range(_L):
            d = c * _L + dd
            wd = wchunk.at[jnp.full((_L,), dd, jnp.int32)].get(
                mode="promise_in_bounds")
            for j in range(8):
                lv = rows_v[d, pl.ds(j * _L, _L)]
                rv = rows_v[D + d, pl.ds(j * _L, _L)]
                out[j] = out[j] + (lv + wd * (rv - lv))
        return tuple(out)

    acc0 = tuple(jnp.zeros((_L,), jnp.float32) for _ in range(8))
    acc = plsc.parallel_loop(0, nchunk, 1, unroll=2, carry=acc0)(c_body)
    for j in range(8):
        ostage_v[0, pl.ds(j * _L, _L)] = acc[j]
    pltpu.sync_copy(ostage_v, out_hbm.at[pl.ds(wid, 1)])


def _tc_body(grid_ref, x_ref, vt_ref, sw_ref, sb_ref, o_ref):
    BTC = x_ref.shape[0] - _NSC
    xc = jnp.clip(x_ref[...], -1.0, 1.0)
    g0 = grid_ref[0]
    inv_h = (_K - 1) / (grid_ref[_K - 1] - g0)
    acc_all = jax.lax.dot_general(xc, sw_ref[...], (((1,), (1,)), ((), ())),
                                  preferred_element_type=jnp.float32)
    acc_all = acc_all + sb_ref[...]
    xt = xc[:BTC]
    u = (xt - g0) * inv_h
    li = u.astype(jnp.int32)
    li = jnp.minimum(li, _K - 2)
    w = u - li.astype(jnp.float32)
    w_bf = w.astype(jnp.bfloat16)
    omw_bf = (1.0 - w).astype(jnp.bfloat16)
    li16 = li.astype(jnp.int16)
    zero_bf = jnp.zeros_like(w_bf)
    acc = acc_all[:BTC]
    eq_prev = li16 == jnp.int16(-1)
    for k in range(_K):
        eq_k = li16 == jnp.int16(k)
        ck = jnp.where(eq_k, omw_bf, jnp.where(eq_prev, w_bf, zero_bf))
        eq_prev = eq_k
        acc = acc + jax.lax.dot(ck, vt_ref[k],
                                preferred_element_type=jnp.float32)
    o_ref[:BTC] = acc
    o_ref[BTC:] = acc_all[BTC:]


def kernel(x, values, skip_w, skip_b, grid):
    B, D = x.shape
    O = values.shape[0]
    vt = jnp.transpose(values, (2, 1, 0))
    tableT = vt.reshape(_K, D, O).transpose(1, 0, 2).reshape(D * _K, O)
    vt_bf = vt.astype(jnp.bfloat16)
    sb = skip_b.reshape(1, O)

    mesh = plsc.VectorSubcoreMesh(core_axis_name="c", subcore_axis_name="s")
    sc_spline = pl.kernel(
        _sc_spline,
        mesh=mesh,
        out_type=jax.ShapeDtypeStruct((_NSC, O), jnp.float32),
        scratch_types=[
            pltpu.VMEM((1, D), jnp.float32),
            pltpu.VMEM((2 * D,), jnp.int32),
            pltpu.VMEM((D,), jnp.float32),
            pltpu.VMEM((2 * D, O), jnp.float32),
            pltpu.VMEM((1, O), jnp.float32),
            pltpu.SemaphoreType.DMA,
        ],
    )(x[B - _NSC:], tableT)

    y_tc = pl.pallas_call(
        _tc_body,
        out_shape=jax.ShapeDtypeStruct((B, O), jnp.float32),
        in_specs=[pl.BlockSpec(memory_space=pltpu.SMEM)]
        + [pl.BlockSpec(memory_space=pltpu.VMEM)] * 4,
        out_specs=pl.BlockSpec(memory_space=pltpu.VMEM),
    )(grid, x, vt_bf, skip_w, sb)

    pad = jnp.zeros((B - _NSC, O), jnp.float32)
    return y_tc + jnp.concatenate([pad, sc_spline], axis=0)


# speedup vs baseline: 27.4320x; 3.8683x over previous
import jax
import jax.numpy as jnp
from jax.experimental import pallas as pl
from jax.experimental.pallas import tpu as pltpu

_K = 16


def _kan_body(grid_ref, x_ref, vt_ref, sw_ref, sb_ref, o_ref):
    xc = jnp.clip(x_ref[...], -1.0, 1.0)
    g0 = grid_ref[0]
    inv_h = (_K - 1) / (grid_ref[_K - 1] - g0)
    u = (xc - g0) * inv_h
    li = u.astype(jnp.int32)
    li = jnp.minimum(li, _K - 2)
    w = u - li.astype(jnp.float32)
    w_bf = w.astype(jnp.bfloat16)
    omw_bf = (1.0 - w).astype(jnp.bfloat16)
    li16 = li.astype(jnp.int16)
    zero_bf = jnp.zeros_like(w_bf)
    acc = jax.lax.dot_general(xc, sw_ref[...], (((1,), (1,)), ((), ())),
                              preferred_element_type=jnp.float32)
    acc = acc + sb_ref[...]
    eq_prev = li16 == jnp.int16(-1)
    for k in range(_K):
        eq_k = li16 == jnp.int16(k)
        ck = jnp.where(eq_k, omw_bf, jnp.where(eq_prev, w_bf, zero_bf))
        eq_prev = eq_k
        acc = acc + jax.lax.dot(ck, vt_ref[k],
                                preferred_element_type=jnp.float32)
    o_ref[...] = acc


def kernel(x, values, skip_w, skip_b, grid):
    B, D = x.shape
    O = values.shape[0]
    vt = jnp.transpose(values, (2, 1, 0)).astype(jnp.bfloat16)
    sb = skip_b.reshape(1, O)
    nt = 4
    bt = B // nt
    return pl.pallas_call(
        _kan_body,
        grid=(nt,),
        out_shape=jax.ShapeDtypeStruct((B, O), jnp.float32),
        in_specs=[
            pl.BlockSpec(memory_space=pltpu.SMEM),
            pl.BlockSpec((bt, D), lambda i: (i, 0)),
            pl.BlockSpec((_K, D, O), lambda i: (0, 0, 0)),
            pl.BlockSpec((O, D), lambda i: (0, 0)),
            pl.BlockSpec((1, O), lambda i: (0, 0)),
        ],
        out_specs=pl.BlockSpec((bt, O), lambda i: (i, 0)),
    )(grid, x, vt, skip_w, sb)


# speedup vs baseline: 29.4202x; 1.0725x over previous
import jax
import jax.numpy as jnp
from jax.experimental import pallas as pl
from jax.experimental.pallas import tpu as pltpu

_K = 16


def _kan_body(grid_ref, x_ref, vt_ref, sw_ref, sb_ref, o_ref):
    xc = jnp.clip(x_ref[...], -1.0, 1.0)
    g0 = grid_ref[0]
    inv_h = (_K - 1) / (grid_ref[_K - 1] - g0)
    u = (xc - g0) * inv_h
    li = u.astype(jnp.int32)
    li = jnp.minimum(li, _K - 2)
    w = u - li.astype(jnp.float32)
    w_bf = w.astype(jnp.bfloat16)
    omw_bf = (1.0 - w).astype(jnp.bfloat16)
    li16 = li.astype(jnp.int16)
    zero_bf = jnp.zeros_like(w_bf)
    acc = jax.lax.dot_general(xc, sw_ref[...], (((1,), (1,)), ((), ())),
                              preferred_element_type=jnp.float32)
    acc = acc + sb_ref[...]
    eq_prev = li16 == jnp.int16(-1)
    for k in range(_K):
        eq_k = li16 == jnp.int16(k)
        ck = jnp.where(eq_k, omw_bf, jnp.where(eq_prev, w_bf, zero_bf))
        eq_prev = eq_k
        acc = acc + jax.lax.dot(ck, vt_ref[k],
                                preferred_element_type=jnp.float32)
    o_ref[...] = acc


def kernel(x, values, skip_w, skip_b, grid):
    B, D = x.shape
    O = values.shape[0]
    vt = jnp.transpose(values, (2, 1, 0)).astype(jnp.bfloat16)
    sb = skip_b.reshape(1, O)
    return pl.pallas_call(
        _kan_body,
        out_shape=jax.ShapeDtypeStruct((B, O), jnp.float32),
        in_specs=[pl.BlockSpec(memory_space=pltpu.SMEM)]
        + [pl.BlockSpec(memory_space=pltpu.VMEM)] * 4,
        out_specs=pl.BlockSpec(memory_space=pltpu.VMEM),
    )(grid, x, vt, skip_w, sb)
